# Initial kernel scaffold; baseline (speedup 1.0000x reference)
import functools

import jax
import jax.numpy as jnp
from jax import lax
from jax.experimental import pallas as pl
from jax.experimental.pallas import tpu as pltpu
from jax.experimental.pallas import tpu_sc as plsc

_N_NODES = 100000
_N_EDGES = 6400000
_NC = 2
_NS = 16
_NW = _NC * _NS
_EPT = _N_EDGES // _NW
_CHUNK = 4000
_NCHUNK = _EPT // _CHUNK


def _newton_sqrt(x):
    i = lax.bitcast_convert_type(x, jnp.int32)
    r = lax.bitcast_convert_type(jnp.int32(0x5F3759DF) - (i >> 1), jnp.float32)
    for _ in range(3):
        r = r * (1.5 - 0.5 * x * r * r)
    return x * r


def _sc_body(rt_h, offt_h, ii_h, jj_h, out_h, table_v, ii_v, jj_v, off_v, acc_v):
    wid = lax.axis_index("s") * _NC + lax.axis_index("c")
    base = wid * _EPT
    for p in range(3):
        pltpu.sync_copy(rt_h.at[p], table_v)

        def chunk_body(k, _, p=p):
            cb = base + k * _CHUNK
            pltpu.sync_copy(ii_h.at[pl.ds(cb, _CHUNK)], ii_v)
            pltpu.sync_copy(jj_h.at[pl.ds(cb, _CHUNK)], jj_v)
            pltpu.sync_copy(offt_h.at[p, pl.ds(cb, _CHUNK)], off_v)
            if p > 0:
                pltpu.sync_copy(out_h.at[pl.ds(cb, _CHUNK)], acc_v)

            def vec_body(v, _, p=p):
                s = v * 16
                ii = ii_v[pl.ds(s, 16)]
                jj = jj_v[pl.ds(s, 16)]
                gi = plsc.load_gather(table_v, [ii])
                gj = plsc.load_gather(table_v, [jj])
                d = gj - gi + off_v[pl.ds(s, 16)]
                sq = d * d
                if p == 0:
                    acc_v[pl.ds(s, 16)] = sq
                elif p == 1:
                    acc_v[pl.ds(s, 16)] = acc_v[pl.ds(s, 16)] + sq
                else:
                    acc_v[pl.ds(s, 16)] = _newton_sqrt(acc_v[pl.ds(s, 16)] + sq)
                return 0

            lax.fori_loop(0, _CHUNK // 16, vec_body, 0, unroll=2)
            pltpu.sync_copy(acc_v, out_h.at[pl.ds(cb, _CHUNK)])
            return 0

        lax.fori_loop(0, _NCHUNK, chunk_body, 0)


@jax.jit
def kernel(R, offsets, idx_i, idx_j):
    rt = R.T
    offt = offsets.T
    mesh = plsc.VectorSubcoreMesh(core_axis_name="c", subcore_axis_name="s")
    f = pl.kernel(
        _sc_body,
        out_type=jax.ShapeDtypeStruct((_N_EDGES,), jnp.float32),
        mesh=mesh,
        scratch_types=[
            pltpu.VMEM((_N_NODES,), jnp.float32),
            pltpu.VMEM((_CHUNK,), jnp.int32),
            pltpu.VMEM((_CHUNK,), jnp.int32),
            pltpu.VMEM((_CHUNK,), jnp.float32),
            pltpu.VMEM((_CHUNK,), jnp.float32),
        ],
    )
    return f(rt, offt, idx_i.astype(jnp.int32), idx_j.astype(jnp.int32))


# speedup vs baseline: 20.8061x; 20.8061x over previous
import functools

import jax
import jax.numpy as jnp
from jax import lax
from jax.experimental import pallas as pl
from jax.experimental.pallas import tpu as pltpu
from jax.experimental.pallas import tpu_sc as plsc

_N_NODES = 100000
_N_EDGES = 6400000
_NC = 2
_NS = 16
_NW = _NC * _NS
_EPT = _N_EDGES // _NW
_CHUNK = 4000
_NCHUNK = _EPT // _CHUNK


def _newton_sqrt(x):
    i = lax.bitcast_convert_type(x, jnp.int32)
    r = lax.bitcast_convert_type(jnp.int32(0x5F3759DF) - (i >> 1), jnp.float32)
    for _ in range(3):
        r = r * (1.5 - 0.5 * x * r * r)
    return x * r


def _sc_body(rt_h, offt_h, ii_h, jj_h, out_h, table_v, ii_v, jj_v, off_v, acc_v):
    wid = lax.axis_index("s") * _NC + lax.axis_index("c")
    base = wid * _EPT
    for p in range(3):
        pltpu.sync_copy(rt_h.at[pl.ds(p * _N_NODES, _N_NODES)], table_v)

        def chunk_body(k, _, p=p):
            cb = base + k * _CHUNK
            pltpu.sync_copy(ii_h.at[pl.ds(cb, _CHUNK)], ii_v)
            pltpu.sync_copy(jj_h.at[pl.ds(cb, _CHUNK)], jj_v)
            pltpu.sync_copy(offt_h.at[pl.ds(p * _N_EDGES + cb, _CHUNK)], off_v)
            if p > 0:
                pltpu.sync_copy(out_h.at[pl.ds(cb, _CHUNK)], acc_v)

            def vec_body(v, _, p=p):
                s = v * 16
                ii = ii_v[pl.ds(s, 16)]
                jj = jj_v[pl.ds(s, 16)]
                gi = plsc.load_gather(table_v, [ii])
                gj = plsc.load_gather(table_v, [jj])
                d = gj - gi + off_v[pl.ds(s, 16)]
                sq = d * d
                if p == 0:
                    acc_v[pl.ds(s, 16)] = sq
                elif p == 1:
                    acc_v[pl.ds(s, 16)] = acc_v[pl.ds(s, 16)] + sq
                else:
                    acc_v[pl.ds(s, 16)] = _newton_sqrt(acc_v[pl.ds(s, 16)] + sq)
                return 0

            lax.fori_loop(0, _CHUNK // 16, vec_body, 0, unroll=2)
            pltpu.sync_copy(acc_v, out_h.at[pl.ds(cb, _CHUNK)])
            return 0

        lax.fori_loop(0, _NCHUNK, chunk_body, 0)


@jax.jit
def kernel(R, offsets, idx_i, idx_j):
    rt = R.T.reshape(-1)
    offt = offsets.T.reshape(-1)
    mesh = plsc.VectorSubcoreMesh(core_axis_name="c", subcore_axis_name="s")
    f = pl.kernel(
        _sc_body,
        out_type=jax.ShapeDtypeStruct((_N_EDGES,), jnp.float32),
        mesh=mesh,
        compiler_params=pltpu.CompilerParams(needs_layout_passes=False),
        scratch_types=[
            pltpu.VMEM((_N_NODES,), jnp.float32),
            pltpu.VMEM((_CHUNK,), jnp.int32),
            pltpu.VMEM((_CHUNK,), jnp.int32),
            pltpu.VMEM((_CHUNK,), jnp.float32),
            pltpu.VMEM((_CHUNK,), jnp.float32),
        ],
    )
    return f(rt, offt, idx_i.astype(jnp.int32), idx_j.astype(jnp.int32))
